# baseline (speedup 1.0000x reference)
import dataclasses
import functools

import jax
import jax.numpy as jnp
from jax import lax
from jax.experimental import pallas as pl
from jax.experimental.pallas import tpu as pltpu
from jax.experimental.pallas import tpu_sc as plsc

N = 10000
E = 320000
D = 128
LANES = 16
EROW = 128
EDGE_ROWS = E // EROW
TROWS = 160
PAD_ROWS = 16 * TROWS
BROWS = 80
CH = 8
NEG_SLOPE = 0.2
NROW0 = 640
NROW15 = 400
NPADS = 10240


def _lrelu(v):
    return jnp.where(v >= 0, v, NEG_SLOPE * v)


def _tc_prep(x, W, a_src, a_dst):
    def body(x_ref, w_ref, as_ref, ad_ref, h_ref, als_ref, ald_ref, c_ref):
        h = lax.dot_general(
            x_ref[...], w_ref[...], (((1,), (0,)), ((), ())),
            precision=lax.Precision.HIGHEST,
            preferred_element_type=jnp.float32)
        h_ref[...] = h
        als = jnp.sum(h * as_ref[...][None, :], axis=1)
        ald = jnp.sum(h * ad_ref[...][None, :], axis=1)
        als_ref[...] = als
        ald_ref[...] = ald
        c = _lrelu(jnp.max(als) + jnp.max(ald))
        c_ref[...] = jnp.full((LANES,), c, jnp.float32)

    return pl.pallas_call(
        body,
        out_shape=[
            jax.ShapeDtypeStruct((N, D), jnp.float32),
            jax.ShapeDtypeStruct((N,), jnp.float32),
            jax.ShapeDtypeStruct((N,), jnp.float32),
            jax.ShapeDtypeStruct((LANES,), jnp.float32),
        ],
    )(x, W, a_src, a_dst)


def _tc_finish(acc, s, h, als, ald, cvec, bias):
    def body(acc_ref, s_ref, h_ref, als_ref, ald_ref, c_ref, b_ref, o_ref):
        sinit = jnp.exp(_lrelu(als_ref[...] + ald_ref[...]) - c_ref[0])
        denom = s_ref[0] + s_ref[1] + sinit
        num = acc_ref[0] + acc_ref[1] + sinit[:, None] * h_ref[...]
        o_ref[...] = num / denom[:, None] + b_ref[...][None, :]

    return pl.pallas_call(
        body,
        out_shape=jax.ShapeDtypeStruct((N, D), jnp.float32),
    )(acc, s, h, als, ald, cvec, bias)


def _sc_logits(src2d, dst2d, als, ald, cvec):
    mesh = plsc.VectorSubcoreMesh(core_axis_name="c", subcore_axis_name="s")
    cp = pltpu.CompilerParams()
    if "needs_layout_passes" in pltpu.CompilerParams.__dataclass_fields__:
        cp = dataclasses.replace(cp, needs_layout_passes=False)

    @functools.partial(
        pl.kernel,
        out_type=[
            jax.ShapeDtypeStruct((PAD_ROWS, EROW), jnp.float32),
            jax.ShapeDtypeStruct((2, NPADS), jnp.float32),
        ],
        mesh=mesh,
        compiler_params=cp,
        scratch_types=[
            pltpu.VMEM((N,), jnp.float32),
            pltpu.VMEM((N,), jnp.float32),
            pltpu.VMEM((CH, EROW), jnp.int32),
            pltpu.VMEM((CH, EROW), jnp.int32),
            pltpu.VMEM((CH, EROW), jnp.float32),
            pltpu.VMEM((EROW,), jnp.float32),
            pltpu.VMEM((LANES,), jnp.float32),
            pltpu.VMEM_SHARED((NPADS,), jnp.float32),
        ],
    )
    def k(src_hbm, dst_hbm, als_hbm, ald_hbm, c_hbm,
          w_hbm, s_hbm, as_v, ad_v, src_c, dst_c, w_c, zbuf, c_v, s_sh):
        cid = lax.axis_index("c")
        sid = lax.axis_index("s")

        pltpu.sync_copy(als_hbm, as_v)
        pltpu.sync_copy(ald_hbm, ad_v)
        pltpu.sync_copy(c_hbm, c_v)

        for kk in range(EROW // LANES):
            zbuf[pl.ds(kk * LANES, LANES)] = jnp.zeros((LANES,), jnp.float32)

        def zero_range(r0, sz):
            @pl.loop(0, sz // BROWS)
            def _(q):
                pltpu.sync_copy(zbuf.at[pl.ds(0, BROWS)],
                                s_sh.at[pl.ds(r0 + q * BROWS, BROWS)])

        @pl.when(sid < 15)
        def _():
            zero_range(sid * NROW0, NROW0)

        @pl.when(sid == 15)
        def _():
            zero_range(15 * NROW0, NROW15)

        plsc.subcore_barrier()

        @pl.loop(0, BROWS // CH)
        def _(cb):
            row0 = sid * TROWS + cid * BROWS + cb * CH
            pltpu.sync_copy(src_hbm.at[pl.ds(row0, CH)], src_c)
            pltpu.sync_copy(dst_hbm.at[pl.ds(row0, CH)], dst_c)

            @pl.loop(0, CH)
            def _(j):
                @pl.when(row0 + j < EDGE_ROWS)
                def _():
                    for kk in range(EROW // LANES):
                        sl = pl.ds(kk * LANES, LANES)
                        es = plsc.load_gather(as_v, [src_c[j, sl]])
                        ed = plsc.load_gather(ad_v, [dst_c[j, sl]])
                        w_c[j, sl] = jnp.exp(_lrelu(es + ed) - c_v[...])
                    pltpu.sync_copy(w_c.at[j], s_sh.at[dst_c.at[j]],
                                    add=True)

                @pl.when(row0 + j >= EDGE_ROWS)
                def _():
                    for kk in range(EROW // LANES):
                        w_c[j, pl.ds(kk * LANES, LANES)] = (
                            jnp.zeros((LANES,), jnp.float32))

            pltpu.sync_copy(w_c, w_hbm.at[pl.ds(row0, CH)])

        plsc.subcore_barrier()

        pltpu.sync_copy(s_sh.at[pl.ds(sid * NROW0, NROW0)],
                        s_hbm.at[cid, pl.ds(sid * NROW0, NROW0)])

    return k(src2d, dst2d, als, ald, cvec)


def _sc_sweep(src2d, dst2d, w2d, h):
    mesh = plsc.VectorSubcoreMesh(core_axis_name="c", subcore_axis_name="s")
    cp = pltpu.CompilerParams()
    if "needs_layout_passes" in pltpu.CompilerParams.__dataclass_fields__:
        cp = dataclasses.replace(cp, needs_layout_passes=False)

    @functools.partial(
        pl.kernel,
        out_type=jax.ShapeDtypeStruct((2, N, D), jnp.float32),
        mesh=mesh,
        compiler_params=cp,
        scratch_types=[
            pltpu.VMEM((CH, EROW), jnp.int32),
            pltpu.VMEM((CH, EROW), jnp.int32),
            pltpu.VMEM((CH, EROW), jnp.float32),
            pltpu.VMEM((EROW,), jnp.float32),
            pltpu.VMEM((2 * EROW, D), jnp.float32),
            pltpu.VMEM_SHARED((N, D), jnp.float32),
            pltpu.SemaphoreType.DMA,
            pltpu.SemaphoreType.DMA,
        ],
    )
    def k(src_hbm, dst_hbm, w_hbm, h_hbm, acc_hbm,
          src_c, dst_c, w_c, wrow, rows2, out_sh, sem0, sem1):
        cid = lax.axis_index("c")
        sid = lax.axis_index("s")
        sems = [sem0, sem1]

        @pl.loop(0, BROWS)
        def _(r):
            for kk in range(D // LANES):
                rows2[r, pl.ds(kk * LANES, LANES)] = (
                    jnp.zeros((LANES,), jnp.float32))

        def zero_range(r0, sz):
            @pl.loop(0, sz // BROWS)
            def _(q):
                pltpu.sync_copy(rows2.at[pl.ds(0, BROWS)],
                                out_sh.at[pl.ds(r0 + q * BROWS, BROWS)])

        @pl.when(sid < 15)
        def _():
            zero_range(sid * NROW0, NROW0)

        @pl.when(sid == 15)
        def _():
            zero_range(15 * NROW0, NROW15)

        plsc.subcore_barrier()

        @pl.loop(0, BROWS // CH)
        def _(cb):
            row0 = sid * TROWS + cid * BROWS + cb * CH
            pltpu.sync_copy(src_hbm.at[pl.ds(row0, CH)], src_c)
            pltpu.sync_copy(dst_hbm.at[pl.ds(row0, CH)], dst_c)
            pltpu.sync_copy(w_hbm.at[pl.ds(row0, CH)], w_c)

            handles = [None, None]
            handles[0] = pltpu.async_copy(
                h_hbm.at[src_c.at[0]], rows2.at[pl.ds(0, EROW)], sems[0])
            for j in range(CH):
                p = j % 2
                handles[p].wait()
                if j < CH - 1:
                    handles[1 - p] = pltpu.async_copy(
                        h_hbm.at[src_c.at[j + 1]],
                        rows2.at[pl.ds((1 - p) * EROW, EROW)], sems[1 - p])
                for kk in range(EROW // LANES):
                    sl = pl.ds(kk * LANES, LANES)
                    wrow[sl] = w_c[j, sl]

                @pl.loop(0, EROW)
                def _(r):
                    splat = plsc.load_gather(
                        wrow, [jnp.full((LANES,), r, jnp.int32)])
                    for kk in range(D // LANES):
                        sl = pl.ds(kk * LANES, LANES)
                        rq = p * EROW + r
                        rows2[rq, sl] = rows2[rq, sl] * splat

                pltpu.sync_copy(rows2.at[pl.ds(p * EROW, EROW)],
                                out_sh.at[dst_c.at[j]], add=True)

        plsc.subcore_barrier()

        @pl.when(sid < 15)
        def _():
            pltpu.sync_copy(out_sh.at[pl.ds(sid * NROW0, NROW0)],
                            acc_hbm.at[cid, pl.ds(sid * NROW0, NROW0)])

        @pl.when(sid == 15)
        def _():
            pltpu.sync_copy(out_sh.at[pl.ds(15 * NROW0, NROW15)],
                            acc_hbm.at[cid, pl.ds(15 * NROW0, NROW15)])

    return k(src2d, dst2d, w2d, h)


def kernel(x, edge_index, W, a_src, a_dst, bias):
    h, als, ald, cvec = _tc_prep(x, W, a_src, a_dst)
    pad = PAD_ROWS * EROW - E
    src2d = jnp.concatenate(
        [edge_index[0], jnp.zeros((pad,), jnp.int32)]).reshape(PAD_ROWS, EROW)
    dst2d = jnp.concatenate(
        [edge_index[1], jnp.zeros((pad,), jnp.int32)]).reshape(PAD_ROWS, EROW)
    w2d, s = _sc_logits(src2d, dst2d, als, ald, cvec)
    acc = _sc_sweep(src2d, dst2d, w2d, h)
    return _tc_finish(acc, s[:, :N], h, als, ald, cvec, bias)


# speedup vs baseline: 1.5651x; 1.5651x over previous
import dataclasses
import functools

import jax
import jax.numpy as jnp
from jax import lax
from jax.experimental import pallas as pl
from jax.experimental.pallas import tpu as pltpu
from jax.experimental.pallas import tpu_sc as plsc

N = 10000
E = 320000
D = 128
LANES = 16
EROW = 128
EDGE_ROWS = E // EROW
TROWS = 160
PAD_ROWS = 16 * TROWS
BROWS = 80
CH = 8
NEG_SLOPE = 0.2
NROW0 = 640
NROW15 = 400
NPADS = 10240


def _lrelu(v):
    return jnp.where(v >= 0, v, NEG_SLOPE * v)


def _tc_prep(x, W, a_src, a_dst):
    def body(x_ref, w_ref, as_ref, ad_ref, h_ref, als_ref, ald_ref, c_ref):
        h = lax.dot_general(
            x_ref[...], w_ref[...], (((1,), (0,)), ((), ())),
            precision=lax.Precision.HIGHEST,
            preferred_element_type=jnp.float32)
        h_ref[...] = h
        als = jnp.sum(h * as_ref[...][None, :], axis=1)
        ald = jnp.sum(h * ad_ref[...][None, :], axis=1)
        als_ref[...] = als
        ald_ref[...] = ald
        c = _lrelu(jnp.max(als) + jnp.max(ald))
        c_ref[...] = jnp.full((LANES,), c, jnp.float32)

    return pl.pallas_call(
        body,
        out_shape=[
            jax.ShapeDtypeStruct((N, D), jnp.float32),
            jax.ShapeDtypeStruct((N,), jnp.float32),
            jax.ShapeDtypeStruct((N,), jnp.float32),
            jax.ShapeDtypeStruct((LANES,), jnp.float32),
        ],
    )(x, W, a_src, a_dst)


def _tc_finish(acc, s, h, als, ald, cvec, bias):
    def body(acc_ref, s_ref, h_ref, als_ref, ald_ref, c_ref, b_ref, o_ref):
        sinit = jnp.exp(_lrelu(als_ref[...] + ald_ref[...]) - c_ref[0])
        denom = s_ref[0] + s_ref[1] + sinit
        num = acc_ref[0] + acc_ref[1] + sinit[:, None] * h_ref[...]
        o_ref[...] = num / denom[:, None] + b_ref[...][None, :]

    return pl.pallas_call(
        body,
        out_shape=jax.ShapeDtypeStruct((N, D), jnp.float32),
    )(acc, s, h, als, ald, cvec, bias)


def _sc_gat(src2d, dst2d, h, als, ald, cvec):
    mesh = plsc.VectorSubcoreMesh(core_axis_name="c", subcore_axis_name="s")
    cp = pltpu.CompilerParams()
    if "needs_layout_passes" in pltpu.CompilerParams.__dataclass_fields__:
        cp = dataclasses.replace(cp, needs_layout_passes=False)

    @functools.partial(
        pl.kernel,
        out_type=[
            jax.ShapeDtypeStruct((2, N, D), jnp.float32),
            jax.ShapeDtypeStruct((2, NPADS), jnp.float32),
        ],
        mesh=mesh,
        compiler_params=cp,
        scratch_types=[
            pltpu.VMEM((N,), jnp.float32),
            pltpu.VMEM((N,), jnp.float32),
            pltpu.VMEM((CH, EROW), jnp.int32),
            pltpu.VMEM((CH, EROW), jnp.int32),
            pltpu.VMEM((EROW,), jnp.float32),
            pltpu.VMEM((EROW, D), jnp.float32),
            pltpu.VMEM((LANES,), jnp.float32),
            pltpu.VMEM_SHARED((NPADS,), jnp.float32),
            pltpu.VMEM_SHARED((N, D), jnp.float32),
            pltpu.SemaphoreType.DMA,
            pltpu.SemaphoreType.DMA,
        ],
    )
    def k(src_hbm, dst_hbm, h_hbm, als_hbm, ald_hbm, c_hbm,
          acc_hbm, s_hbm, as_v, ad_v, src_c, dst_c, wrow, rows_v,
          c_v, s_sh, out_sh, sem_a, sem_b):
        cid = lax.axis_index("c")
        sid = lax.axis_index("s")

        pltpu.sync_copy(als_hbm, as_v)
        pltpu.sync_copy(ald_hbm, ad_v)
        pltpu.sync_copy(c_hbm, c_v)

        @pl.loop(0, EROW)
        def _(r):
            for kk in range(D // LANES):
                rows_v[r, pl.ds(kk * LANES, LANES)] = (
                    jnp.zeros((LANES,), jnp.float32))
        for kk in range(EROW // LANES):
            wrow[pl.ds(kk * LANES, LANES)] = jnp.zeros((LANES,), jnp.float32)

        def zero_range(r0, sz):
            @pl.loop(0, sz // BROWS)
            def _(q):
                q0 = r0 + q * BROWS
                pltpu.sync_copy(rows_v.at[pl.ds(0, BROWS)],
                                out_sh.at[pl.ds(q0, BROWS)])
                pltpu.sync_copy(wrow.at[pl.ds(0, BROWS)],
                                s_sh.at[pl.ds(q0, BROWS)])

        @pl.when(sid < 15)
        def _():
            zero_range(sid * NROW0, NROW0)

        @pl.when(sid == 15)
        def _():
            zero_range(15 * NROW0, NROW15)

        plsc.subcore_barrier()

        @pl.loop(0, BROWS // CH)
        def _(cb):
            row0 = sid * TROWS + cid * BROWS + cb * CH
            pltpu.sync_copy(src_hbm.at[pl.ds(row0, CH)], src_c)
            pltpu.sync_copy(dst_hbm.at[pl.ds(row0, CH)], dst_c)

            @pl.loop(0, CH)
            def _(j):
                @pl.when(row0 + j < EDGE_ROWS)
                def _():
                    HALF = EROW // 2
                    ca = pltpu.async_copy(
                        h_hbm.at[src_c.at[j, pl.ds(0, HALF)]],
                        rows_v.at[pl.ds(0, HALF)], sem_a)
                    cb = pltpu.async_copy(
                        h_hbm.at[src_c.at[j, pl.ds(HALF, HALF)]],
                        rows_v.at[pl.ds(HALF, HALF)], sem_b)

                    for kk in range(EROW // LANES):
                        sl = pl.ds(kk * LANES, LANES)
                        sv = src_c[j, sl]
                        dv = dst_c[j, sl]
                        es = plsc.load_gather(as_v, [sv])
                        ed = plsc.load_gather(ad_v, [dv])
                        e = _lrelu(es + ed)
                        wrow[sl] = jnp.exp(e - c_v[...])
                    pltpu.sync_copy(wrow, s_sh.at[dst_c.at[j]], add=True)

                    def scale(r0):
                        @pl.loop(r0, r0 + HALF)
                        def _(r):
                            splat = plsc.load_gather(
                                wrow, [jnp.full((LANES,), r, jnp.int32)])
                            for kk in range(D // LANES):
                                sl = pl.ds(kk * LANES, LANES)
                                rows_v[r, sl] = rows_v[r, sl] * splat

                    ca.wait()
                    scale(0)
                    cb.wait()
                    scale(HALF)

                    pltpu.sync_copy(rows_v, out_sh.at[dst_c.at[j]],
                                    add=True)

        plsc.subcore_barrier()

        pltpu.sync_copy(s_sh.at[pl.ds(sid * NROW0, NROW0)],
                        s_hbm.at[cid, pl.ds(sid * NROW0, NROW0)])

        @pl.when(sid < 15)
        def _():
            pltpu.sync_copy(out_sh.at[pl.ds(sid * NROW0, NROW0)],
                            acc_hbm.at[cid, pl.ds(sid * NROW0, NROW0)])

        @pl.when(sid == 15)
        def _():
            pltpu.sync_copy(out_sh.at[pl.ds(15 * NROW0, NROW15)],
                            acc_hbm.at[cid, pl.ds(15 * NROW0, NROW15)])

    return k(src2d, dst2d, h, als, ald, cvec)


def kernel(x, edge_index, W, a_src, a_dst, bias):
    h, als, ald, cvec = _tc_prep(x, W, a_src, a_dst)
    pad = PAD_ROWS * EROW - E
    src2d = jnp.concatenate(
        [edge_index[0], jnp.zeros((pad,), jnp.int32)]).reshape(PAD_ROWS, EROW)
    dst2d = jnp.concatenate(
        [edge_index[1], jnp.zeros((pad,), jnp.int32)]).reshape(PAD_ROWS, EROW)
    acc, s = _sc_gat(src2d, dst2d, h, als, ald, cvec)
    return _tc_finish(acc, s[:, :N], h, als, ald, cvec, bias)
